# baseline (speedup 1.0000x reference)
import functools

import jax
import jax.numpy as jnp
from jax import lax
from jax.experimental import pallas as pl
from jax.experimental.pallas import tpu as pltpu

B = 1024
N = 32768
F = 96
D = 256
DI = 512
NC = 10
CTX = 96


def _doth(a, b):
    return jnp.dot(a, b, preferred_element_type=jnp.float32,
                   precision=lax.Precision.HIGHEST)


def _dotb(a, b):
    return jnp.dot(a.astype(jnp.bfloat16), b.astype(jnp.bfloat16),
                   preferred_element_type=jnp.float32)


def _encoder_body(rows_ref, wlin_ref, blin_ref, we0_ref, be0_ref, we1_ref,
                  be1_ref, lng_ref, lnb_ref, wk_ref, bk_ref, z_ref, key_ref):
    rows = rows_ref[...]
    z = _dotb(rows, wlin_ref[...])
    z = z + blin_ref[...]
    h = _dotb(z, we0_ref[...])
    h = jnp.maximum(h + be0_ref[...], 0.0)
    z = z + _dotb(h, we1_ref[...]) + be1_ref[...]
    z_ref[...] = z
    m = jnp.mean(z, axis=-1, keepdims=True)
    v = jnp.mean((z - m) * (z - m), axis=-1, keepdims=True)
    zn = (z - m) / jnp.sqrt(v + 1e-5) * lng_ref[...] + lnb_ref[...]
    key_ref[...] = _dotb(zn, wk_ref[...]) + bk_ref[...]


def _encode(rows, blk, wlin, blin, we0, be0, we1, be1, lng, lnb, wk, bk):
    R = rows.shape[0]
    grid = (R // blk,)
    full = lambda s: pl.BlockSpec(s, lambda i: (0, 0))
    z, key = pl.pallas_call(
        _encoder_body,
        grid=grid,
        in_specs=[
            pl.BlockSpec((blk, F), lambda i: (i, 0)),
            full((F, D)), full((1, D)), full((D, DI)), full((1, DI)),
            full((DI, D)), full((1, D)), full((1, D)), full((1, D)),
            full((D, D)), full((1, D)),
        ],
        out_specs=[pl.BlockSpec((blk, D), lambda i: (i, 0)),
                   pl.BlockSpec((blk, D), lambda i: (i, 0))],
        out_shape=[jax.ShapeDtypeStruct((R, D), jnp.float32),
                   jax.ShapeDtypeStruct((R, D), jnp.float32)],
    )(rows, wlin, blin, we0, be0, we1, be1, lng, lnb, wk, bk)
    return z, key


def _dist_body(q_ref, kn_ref, ones_ref, dist_ref):
    q = q_ref[...]
    kn = kn_ref[...]
    qn = jnp.sum(q * q, axis=-1, keepdims=True)
    qk = lax.dot_general(q.astype(jnp.bfloat16), kn.astype(jnp.bfloat16),
                         (((1,), (1,)), ((), ())),
                         preferred_element_type=jnp.float32)
    nn = lax.dot_general(ones_ref[...], kn * kn, (((1,), (1,)), ((), ())),
                         preferred_element_type=jnp.float32,
                         precision=lax.Precision.HIGHEST)
    dist_ref[...] = (qn - 2.0 * qk) + nn[0:1, :]


def _distances(qkey, nkey):
    QB, NB = 256, 4096
    ones = jnp.ones((8, D), jnp.float32)
    return pl.pallas_call(
        _dist_body,
        grid=(B // QB, N // NB),
        in_specs=[
            pl.BlockSpec((QB, D), lambda i, j: (i, 0)),
            pl.BlockSpec((NB, D), lambda i, j: (j, 0)),
            pl.BlockSpec((8, D), lambda i, j: (0, 0)),
        ],
        out_specs=pl.BlockSpec((QB, NB), lambda i, j: (i, j)),
        out_shape=jax.ShapeDtypeStruct((B, N), jnp.float32),
    )(qkey, nkey, ones)


def _tail_body(x_ref, k_ref, ki_ref, y_ref, emb_ref, wt0_ref, bt0_ref,
               wt1_ref, lnpg_ref, lnpb_ref, wp0_ref, bp0_ref, wp1_ref,
               bp1_ref, lnPg_ref, lnPb_ref, wP_ref, bP_ref, out_ref):
    QB = x_ref.shape[0]
    R = QB * CTX

    sub_i = lax.broadcasted_iota(jnp.int32, (R, QB), 0)
    lane_i = lax.broadcasted_iota(jnp.int32, (R, QB), 1)
    m1 = jnp.where(sub_i // CTX == lane_i, 1.0, 0.0)
    sub_c = lax.broadcasted_iota(jnp.int32, (R, CTX), 0)
    lane_c = lax.broadcasted_iota(jnp.int32, (R, CTX), 1)
    m2 = jnp.where(sub_c % CTX == lane_c, 1.0, 0.0)
    seg_s = lax.broadcasted_iota(jnp.int32, (QB, R), 0)
    seg_l = lax.broadcasted_iota(jnp.int32, (QB, R), 1)
    seg = jnp.where(seg_l // CTX == seg_s, 1.0, 0.0)

    kexp = _doth(m1, k_ref[...])
    ki = ki_ref[...]

    kb = kexp.astype(jnp.bfloat16).astype(jnp.float32)
    kib = ki.astype(jnp.bfloat16).astype(jnp.float32)
    cross = jnp.sum(kb * kib, axis=-1, keepdims=True)
    qn = jnp.sum(kexp * kexp, axis=-1, keepdims=True)
    kn2 = jnp.sum(ki * ki, axis=-1, keepdims=True)
    s_col = (qn - 2.0 * cross) + kn2
    s = _doth(seg, s_col * m2)
    w = jnp.exp(s - jnp.max(s, axis=-1, keepdims=True))
    w = w / jnp.sum(w, axis=-1, keepdims=True)

    y = y_ref[...]
    yexp = _doth(m1, y.astype(jnp.float32))
    y_col = jnp.sum(yexp * m2, axis=-1, keepdims=True)
    emb_rows = jnp.zeros((R, D), jnp.float32)
    for c in range(NC):
        emb_rows = emb_rows + jnp.where(y_col == c, 1.0, 0.0) * emb_ref[c:c + 1, :]

    wexp = _doth(m1, w)
    wcol = jnp.sum(wexp * m2, axis=-1, keepdims=True)

    diff = kexp - ki
    h = _dotb(diff, wt0_ref[...])
    h = jnp.maximum(h + bt0_ref[...], 0.0)
    vrows = emb_rows + _dotb(h, wt1_ref[...])
    v16 = vrows.astype(jnp.bfloat16).astype(jnp.float32)
    w16 = wcol.astype(jnp.bfloat16).astype(jnp.float32)
    V = _doth(seg, w16 * v16)
    xn = x_ref[...] + V

    m = jnp.mean(xn, axis=-1, keepdims=True)
    v = jnp.mean((xn - m) * (xn - m), axis=-1, keepdims=True)
    xl = (xn - m) / jnp.sqrt(v + 1e-5) * lnpg_ref[...] + lnpb_ref[...]
    h2 = _doth(xl, wp0_ref[...])
    h2 = jnp.maximum(h2 + bp0_ref[...], 0.0)
    xn = xn + _doth(h2, wp1_ref[...]) + bp1_ref[...]

    m = jnp.mean(xn, axis=-1, keepdims=True)
    v = jnp.mean((xn - m) * (xn - m), axis=-1, keepdims=True)
    xl = (xn - m) / jnp.sqrt(v + 1e-5) * lnPg_ref[...] + lnPb_ref[...]
    xl = jnp.maximum(xl, 0.0)
    out_ref[...] = _doth(xl, wP_ref[...]) + bP_ref[...]


def _tail(x, k, ki_rows, y_sel, emb_y, wt0, bt0, wt1, lnpg, lnpb, wp0,
          bp0, wp1, bp1, lnPg, lnPb, wP, bP):
    QB = 32
    R = QB * CTX
    grid = (B // QB,)
    full = lambda s: pl.BlockSpec(s, lambda i: tuple(0 for _ in s))
    return pl.pallas_call(
        _tail_body,
        grid=grid,
        in_specs=[
            pl.BlockSpec((QB, D), lambda i: (i, 0)),
            pl.BlockSpec((QB, D), lambda i: (i, 0)),
            pl.BlockSpec((R, D), lambda i: (i, 0)),
            pl.BlockSpec((QB, CTX), lambda i: (i, 0)),
            full((NC, D)), full((D, DI)), full((1, DI)), full((DI, D)),
            full((1, D)), full((1, D)), full((D, DI)), full((1, DI)),
            full((DI, D)), full((1, D)), full((1, D)), full((1, D)),
            full((D, 128)), full((1, 128)),
        ],
        out_specs=pl.BlockSpec((QB, 128), lambda i: (i, 0)),
        out_shape=jax.ShapeDtypeStruct((B, 128), jnp.float32),
    )(x, k, ki_rows, y_sel, emb_y, wt0, bt0, wt1, lnpg, lnpb, wp0, bp0,
      wp1, bp1, lnPg, lnPb, wP, bP)


def kernel(x_num, candidat_x_num, candidat_y, context_size, W_lin, b_lin,
           W_e0, b_e0, W_e1, b_e1, ln_g, ln_b, W_k, b_k, emb_y, W_t0, b_t0,
           W_t1, lnp_g, lnp_b, W_p0, b_p0, W_p1, b_p1, lnP_g, lnP_b, W_P,
           b_P):
    r2 = lambda a: a.reshape(1, -1)
    blin, be0, be1, lng, lnb, bk = map(r2, (b_lin, b_e0, b_e1, ln_g, ln_b, b_k))
    bt0, lnpg, lnpb, bp0, bp1 = map(r2, (b_t0, lnp_g, lnp_b, b_p0, b_p1))
    lnPg, lnPb, bP = map(r2, (lnP_g, lnP_b, b_P))

    x, k = _encode(x_num, 1024, W_lin, blin, W_e0, be0, W_e1, be1, lng, lnb,
                   W_k, bk)
    _, nkey = _encode(candidat_x_num, 2048, W_lin, blin, W_e0, be0, W_e1,
                      be1, lng, lnb, W_k, bk)

    dist = _distances(k, nkey)

    _, I = lax.top_k(-dist, CTX)

    ki_rows = nkey[I.reshape(-1)]
    y_sel = candidat_y[I].astype(jnp.int32)

    wPp = jnp.zeros((D, 128), jnp.float32).at[:, :NC].set(W_P)
    bPp = jnp.zeros((1, 128), jnp.float32).at[:, :NC].set(bP)
    out = _tail(x, k, ki_rows, y_sel, emb_y, W_t0, bt0, W_t1, lnpg, lnpb,
                W_p0, bp0, W_p1, bp1, lnPg, lnPb, wPp, bPp)
    return out[:, :NC]
